# baseline (speedup 1.0000x reference)
import functools

import jax
import jax.numpy as jnp
from jax import lax
from jax.experimental import pallas as pl
from jax.experimental.pallas import tpu as pltpu
from jax.experimental.pallas import tpu_sc as plsc

_NC = 2
_NS = 16
_NW = _NC * _NS
_CHUNK = 32768


def _make_sc_add(total, pos_total):
    per_w = total // _NW
    n_chunks = per_w // _CHUNK
    mesh = plsc.VectorSubcoreMesh(core_axis_name="c", subcore_axis_name="s")

    @functools.partial(
        pl.kernel,
        mesh=mesh,
        out_type=jax.ShapeDtypeStruct((total,), jnp.float32),
        scratch_types=[
            pltpu.VMEM((_CHUNK,), jnp.float32),
            pltpu.VMEM((_CHUNK,), jnp.float32),
        ],
    )
    def sc_add(x_hbm, pos_hbm, out_hbm, xbuf, pbuf):
        wid = lax.axis_index("s") * _NC + lax.axis_index("c")
        base = wid * per_w

        def chunk_body(j, carry):
            e = base + j * _CHUNK
            pe = lax.rem(e, pos_total)
            pltpu.sync_copy(x_hbm.at[pl.ds(e, _CHUNK)], xbuf)
            pltpu.sync_copy(pos_hbm.at[pl.ds(pe, _CHUNK)], pbuf)

            def add_body(i, c):
                off = i * 128
                for u in range(8):
                    s = pl.ds(off + u * 16, 16)
                    xbuf[s] = xbuf[s] + pbuf[s]
                return c

            lax.fori_loop(0, _CHUNK // 128, add_body, 0, unroll=False)
            pltpu.sync_copy(xbuf, out_hbm.at[pl.ds(e, _CHUNK)])
            return carry

        lax.fori_loop(0, n_chunks, chunk_body, 0, unroll=False)

    return sc_add


@jax.jit
def kernel(x, pos_table):
    batch, seq_len, d = x.shape
    total = batch * seq_len * d
    pos_total = seq_len * d
    out = _make_sc_add(total, pos_total)(
        x.reshape(total), pos_table[:seq_len].reshape(pos_total)
    )
    return out.reshape(batch, seq_len, d)


# speedup vs baseline: 1.4337x; 1.4337x over previous
import functools

import jax
import jax.numpy as jnp
from jax import lax
from jax.experimental import pallas as pl
from jax.experimental.pallas import tpu as pltpu
from jax.experimental.pallas import tpu_sc as plsc

_NC = 2
_NS = 16
_NW = _NC * _NS
_CHUNK = 32768


def _make_sc_add(total, x_base, pos_base):
    per_w = total // _NW
    n_chunks = per_w // _CHUNK
    mesh = plsc.VectorSubcoreMesh(core_axis_name="c", subcore_axis_name="s")

    @functools.partial(
        pl.kernel,
        mesh=mesh,
        out_type=jax.ShapeDtypeStruct((total,), jnp.float32),
        scratch_types=[
            pltpu.VMEM((_CHUNK,), jnp.float32),
            pltpu.VMEM((_CHUNK,), jnp.float32),
        ],
    )
    def sc_add(x_hbm, pos_hbm, out_hbm, xbuf, pbuf):
        wid = lax.axis_index("s") * _NC + lax.axis_index("c")
        base = wid * per_w

        def chunk_body(j, carry):
            l = base + j * _CHUNK
            pltpu.sync_copy(x_hbm.at[pl.ds(x_base + l, _CHUNK)], xbuf)
            pltpu.sync_copy(pos_hbm.at[pl.ds(pos_base + l, _CHUNK)], pbuf)

            def add_body(i, c):
                off = i * 128
                for u in range(8):
                    s = pl.ds(off + u * 16, 16)
                    xbuf[s] = xbuf[s] + pbuf[s]
                return c

            lax.fori_loop(0, _CHUNK // 128, add_body, 0, unroll=False)
            pltpu.sync_copy(xbuf, out_hbm.at[pl.ds(l, _CHUNK)])
            return carry

        lax.fori_loop(0, n_chunks, chunk_body, 0, unroll=False)

    return sc_add


def _add_rows_block(x_ref, pos_ref, o_ref):
    o_ref[...] = x_ref[...] + pos_ref[...]


def _tc_add_rows(x2d, pos, r_tc, s_blk=512):
    seq_len, d = pos.shape
    n_pos_blocks = seq_len // s_blk
    return pl.pallas_call(
        _add_rows_block,
        grid=(r_tc // s_blk,),
        in_specs=[
            pl.BlockSpec((s_blk, d), lambda r: (r, 0)),
            pl.BlockSpec((s_blk, d), lambda r: (lax.rem(r, n_pos_blocks), 0)),
        ],
        out_specs=pl.BlockSpec((s_blk, d), lambda r: (r, 0)),
        out_shape=jax.ShapeDtypeStruct((r_tc, d), x2d.dtype),
    )(x2d, pos)


@jax.jit
def kernel(x, pos_table):
    batch, seq_len, d = x.shape
    rows = batch * seq_len
    r_tc = (rows * 7) // 8
    r_sc = rows - r_tc
    x2d = x.reshape(rows, d)
    pos = pos_table[:seq_len]
    tc_out = _tc_add_rows(x2d, pos, r_tc)
    sc_total = r_sc * d
    pos_base = (r_tc % seq_len) * d
    sc_out = _make_sc_add(sc_total, r_tc * d, pos_base)(
        x2d.reshape(rows * d), pos.reshape(seq_len * d)
    )
    out2d = jnp.concatenate([tc_out, sc_out.reshape(r_sc, d)], axis=0)
    return out2d.reshape(batch, seq_len, d)
